# Initial kernel scaffold; baseline (speedup 1.0000x reference)
import jax
import jax.numpy as jnp
from jax.experimental import pallas as pl


def kernel(x, edge_index, edge_weight, W1, b1, W2, b2):
    raise NotImplementedError("write your pallas kernel here")


# speedup vs baseline: 9.0999x; 9.0999x over previous
import functools

import jax
import jax.numpy as jnp
from jax import lax
from jax.experimental import pallas as pl
from jax.experimental.pallas import tpu as pltpu
from jax.experimental.pallas import tpu_sc as plsc

N = 10000
E = 320000
D = 128

NC = 2
NS = 16
L = 16
NW = NC * NS
EPW = E // NW
C = 80
NCHUNK = EPW // C
N2 = 10240
RPT = N2 // NS
ZR = 128


def _sc_mesh():
    return plsc.VectorSubcoreMesh(core_axis_name="c", subcore_axis_name="s",
                                  num_cores=NC, num_subcores=NS)


def _sc_deg_body(dst_hbm, ew_hbm, out_hbm, acc_sh, dstv, ewv, msg, zbuf):
    cid = lax.axis_index("c")
    sid = lax.axis_index("s")
    wid = sid * NC + cid
    base = wid * EPW
    row0 = sid * RPT

    lanes0 = jnp.zeros((L,), jnp.float32)

    def zb(j, _):
        zbuf[j, :] = lanes0
        return 0
    lax.fori_loop(0, RPT, zb, 0)
    pltpu.sync_copy(zbuf, acc_sh.at[pl.ds(row0, RPT)])
    plsc.subcore_barrier()

    iota = lax.iota(jnp.int32, L)
    e0 = jnp.where(iota == 0, 1.0, 0.0).astype(jnp.float32)

    def chunk(c, _):
        b = base + c * C
        pltpu.sync_copy(dst_hbm.at[pl.ds(b, C)], dstv)
        pltpu.sync_copy(ew_hbm.at[pl.ds(b, C)], ewv)

        def mrow(gi, _):
            j0 = gi * L
            ew16 = ewv[pl.ds(j0, L)]
            for j in range(L):
                msg[j0 + j, :] = ew16[j] * e0
            return 0
        lax.fori_loop(0, C // L, mrow, 0)
        pltpu.sync_copy(msg, acc_sh.at[dstv], add=True)
        return 0

    lax.fori_loop(0, NCHUNK, chunk, 0)

    plsc.subcore_barrier()
    pltpu.sync_copy(acc_sh.at[pl.ds(row0, RPT)],
                    out_hbm.at[pl.ds(cid * N2 + row0, RPT)])


def _sc_deg(dst, ew):
    k = pl.kernel(
        _sc_deg_body,
        out_type=jax.ShapeDtypeStruct((NC * N2, L), jnp.float32),
        mesh=_sc_mesh(),
        scratch_types=[
            pltpu.VMEM_SHARED((N2, L), jnp.float32),
            pltpu.VMEM((C,), jnp.int32),
            pltpu.VMEM((C,), jnp.float32),
            pltpu.VMEM((C, L), jnp.float32),
            pltpu.VMEM((RPT, L), jnp.float32),
        ],
    )
    return k(dst, ew)


def _sc_agg_body(g_hbm, src_hbm, dst_hbm, ew_hbm, out_hbm,
                 acc_sh, srcv, dstv, ewv, rows, zbuf, gsem):
    cid = lax.axis_index("c")
    sid = lax.axis_index("s")
    wid = sid * NC + cid
    base = wid * EPW
    row0 = sid * RPT

    lanes0 = jnp.zeros((L,), jnp.float32)

    def zb(j, _):
        for kk in range(D // L):
            zbuf[j, pl.ds(kk * L, L)] = lanes0
        return 0
    lax.fori_loop(0, ZR, zb, 0)
    for i in range(RPT // ZR):
        pltpu.sync_copy(zbuf, acc_sh.at[pl.ds(row0 + i * ZR, ZR)])
    plsc.subcore_barrier()

    def chunk(c, _):
        b = base + c * C
        pltpu.sync_copy(src_hbm.at[pl.ds(b, C)], srcv)
        pltpu.sync_copy(dst_hbm.at[pl.ds(b, C)], dstv)
        pltpu.sync_copy(ew_hbm.at[pl.ds(b, C)], ewv)
        pltpu.async_copy(g_hbm.at[srcv], rows, gsem).wait()

        def scale(gi, _):
            j0 = gi * L
            ew16 = ewv[pl.ds(j0, L)]
            for j in range(L):
                w = ew16[j]
                for kk in range(D // L):
                    rows[j0 + j, pl.ds(kk * L, L)] = (
                        rows[j0 + j, pl.ds(kk * L, L)] * w)
            return 0
        lax.fori_loop(0, C // L, scale, 0)

        pltpu.sync_copy(rows, acc_sh.at[dstv], add=True)
        return 0

    lax.fori_loop(0, NCHUNK, chunk, 0)

    plsc.subcore_barrier()
    for i in range(RPT // ZR):
        pltpu.sync_copy(acc_sh.at[pl.ds(row0 + i * ZR, ZR)],
                        out_hbm.at[pl.ds(cid * N2 + row0 + i * ZR, ZR)])


def _sc_agg(g, src, dst, ew):
    k = pl.kernel(
        _sc_agg_body,
        out_type=jax.ShapeDtypeStruct((NC * N2, D), jnp.float32),
        mesh=_sc_mesh(),
        scratch_types=[
            pltpu.VMEM_SHARED((N2, D), jnp.float32),
            pltpu.VMEM((C,), jnp.int32),
            pltpu.VMEM((C,), jnp.int32),
            pltpu.VMEM((C,), jnp.float32),
            pltpu.VMEM((C, D), jnp.float32),
            pltpu.VMEM((ZR, D), jnp.float32),
            pltpu.SemaphoreType.DMA,
        ],
    )
    return k(g, src, dst, ew)


BM = 256


def _tc1_body(d0_ref, d1_ref, x_ref, w_ref, g_ref, dinv_ref):
    s = jnp.sum(d0_ref[...] + d1_ref[...], axis=1, keepdims=True) + 1.0
    dv = lax.rsqrt(s)
    t = jnp.dot(x_ref[...], w_ref[...], preferred_element_type=jnp.float32)
    g_ref[...] = dv * t
    dinv_ref[...] = dv


def _tc1(degp0, degp1, x, W1):
    grid = (pl.cdiv(N, BM),)
    return pl.pallas_call(
        _tc1_body,
        grid=grid,
        in_specs=[
            pl.BlockSpec((BM, L), lambda i: (i, 0)),
            pl.BlockSpec((BM, L), lambda i: (i, 0)),
            pl.BlockSpec((BM, D), lambda i: (i, 0)),
            pl.BlockSpec((D, D), lambda i: (0, 0)),
        ],
        out_specs=[
            pl.BlockSpec((BM, D), lambda i: (i, 0)),
            pl.BlockSpec((BM, 1), lambda i: (i, 0)),
        ],
        out_shape=[
            jax.ShapeDtypeStruct((N, D), jnp.float32),
            jax.ShapeDtypeStruct((N, 1), jnp.float32),
        ],
    )(degp0, degp1, x, W1)


def _tc2_body(p0_ref, p1_ref, g_ref, dv_ref, b_ref, w_ref, g2_ref):
    dv = dv_ref[...]
    h = jnp.maximum(dv * (p0_ref[...] + p1_ref[...] + g_ref[...]) + b_ref[...],
                    0.0)
    t2 = jnp.dot(h, w_ref[...], preferred_element_type=jnp.float32)
    g2_ref[...] = dv * t2


def _tc2(p0, p1, g1, dinv, b1, W2):
    grid = (pl.cdiv(N, BM),)
    return pl.pallas_call(
        _tc2_body,
        grid=grid,
        in_specs=[
            pl.BlockSpec((BM, D), lambda i: (i, 0)),
            pl.BlockSpec((BM, D), lambda i: (i, 0)),
            pl.BlockSpec((BM, D), lambda i: (i, 0)),
            pl.BlockSpec((BM, 1), lambda i: (i, 0)),
            pl.BlockSpec((1, D), lambda i: (0, 0)),
            pl.BlockSpec((D, D), lambda i: (0, 0)),
        ],
        out_specs=pl.BlockSpec((BM, D), lambda i: (i, 0)),
        out_shape=jax.ShapeDtypeStruct((N, D), jnp.float32),
    )(p0, p1, g1, dinv, b1, W2)


def _tc3_body(p0_ref, p1_ref, g_ref, dv_ref, b_ref, out_ref):
    dv = dv_ref[...]
    out_ref[...] = jnp.maximum(
        dv * (p0_ref[...] + p1_ref[...] + g_ref[...]) + b_ref[...], 0.0)


def _tc3(p0, p1, g2, dinv, b2):
    grid = (pl.cdiv(N, BM),)
    return pl.pallas_call(
        _tc3_body,
        grid=grid,
        in_specs=[
            pl.BlockSpec((BM, D), lambda i: (i, 0)),
            pl.BlockSpec((BM, D), lambda i: (i, 0)),
            pl.BlockSpec((BM, D), lambda i: (i, 0)),
            pl.BlockSpec((BM, 1), lambda i: (i, 0)),
            pl.BlockSpec((1, D), lambda i: (0, 0)),
        ],
        out_specs=pl.BlockSpec((BM, D), lambda i: (i, 0)),
        out_shape=jax.ShapeDtypeStruct((N, D), jnp.float32),
    )(p0, p1, g2, dinv, b2)


@jax.jit
def _run(x, edge_index, edge_weight, W1, b1, W2, b2):
    src = edge_index[0]
    dst = edge_index[1]
    b1r = b1.reshape(1, D)
    b2r = b2.reshape(1, D)

    degp = _sc_deg(dst, edge_weight)
    g1, dinv = _tc1(degp[:N], degp[N2:N2 + N], x, W1)

    s1 = _sc_agg(g1, src, dst, edge_weight)
    g2 = _tc2(s1[:N], s1[N2:N2 + N], g1, dinv, b1r, W2)

    s2 = _sc_agg(g2, src, dst, edge_weight)
    h2 = _tc3(s2[:N], s2[N2:N2 + N], g2, dinv, b2r)
    return h2


def kernel(x, edge_index, edge_weight, W1, b1, W2, b2):
    return (_run(x, edge_index, edge_weight, W1, b1, W2, b2), None)


# speedup vs baseline: 18.4250x; 2.0248x over previous
import functools

import jax
import jax.numpy as jnp
from jax import lax
from jax.experimental import pallas as pl
from jax.experimental.pallas import tpu as pltpu
from jax.experimental.pallas import tpu_sc as plsc

N = 10000
E = 320000
D = 128

NC = 2
NS = 16
L = 16
NW = NC * NS
EPW = E // NW
C = 80
NCHUNK = EPW // C
N2 = 10240
RPT = N2 // NS
ZR = 128


def _sc_mesh():
    return plsc.VectorSubcoreMesh(core_axis_name="c", subcore_axis_name="s",
                                  num_cores=NC, num_subcores=NS)


def _sc_deg_body(dst_hbm, ew_hbm, out_hbm, acc_sh, dstv0, dstv1, ew_all,
                 msg0, msg1, zbuf, ssem0, ssem1, isem0, isem1):
    cid = lax.axis_index("c")
    sid = lax.axis_index("s")
    wid = sid * NC + cid
    row0 = sid * RPT

    lanes0 = jnp.zeros((L,), jnp.float32)

    def zb(j, _):
        zbuf[j, :] = lanes0
        return 0
    lax.fori_loop(0, RPT, zb, 0)
    pltpu.sync_copy(zbuf, acc_sh.at[pl.ds(row0, RPT)])
    pltpu.sync_copy(ew_hbm.at[wid], ew_all)
    plsc.subcore_barrier()

    base = wid * EPW
    iota = lax.iota(jnp.int32, L)
    e0 = jnp.where(iota == 0, 1.0, 0.0).astype(jnp.float32)

    def start_idx(c, dstv, sem):
        pltpu.async_copy(dst_hbm.at[pl.ds(base + c * C, C)], dstv, sem)

    def wait_idx(c, dstv, sem):
        pltpu.make_async_copy(dst_hbm.at[pl.ds(base + c * C, C)], dstv,
                              sem).wait()

    def build(c, msg):
        def mrow(gi, _):
            j0 = gi * L
            ew16 = ew_all[pl.ds(c * C + j0, L)]
            for j in range(L):
                msg[j0 + j, :] = ew16[j] * e0
            return 0
        lax.fori_loop(0, C // L, mrow, 0)

    def start_scatter(msg, dstv, sem):
        pltpu.async_copy(msg, acc_sh.at[dstv], sem, add=True)

    def wait_scatter(msg, dstv, sem):
        pltpu.make_async_copy(msg, acc_sh.at[dstv], sem).wait()

    start_idx(0, dstv0, isem0)
    start_idx(1, dstv1, isem1)

    def pair(i, _):
        a = 2 * i
        b = a + 1
        build(a, msg0)
        wait_idx(a, dstv0, isem0)
        start_scatter(msg0, dstv0, ssem0)
        build(b, msg1)
        wait_idx(b, dstv1, isem1)
        start_scatter(msg1, dstv1, ssem1)
        wait_scatter(msg0, dstv0, ssem0)
        start_idx(a + 2, dstv0, isem0)
        wait_scatter(msg1, dstv1, ssem1)

        @pl.when(b + 2 < NCHUNK)
        def _():
            start_idx(b + 2, dstv1, isem1)
        return 0

    lax.fori_loop(0, NCHUNK // 2, pair, 0)
    last = NCHUNK - 1
    build(last, msg0)
    wait_idx(last, dstv0, isem0)
    start_scatter(msg0, dstv0, ssem0)
    wait_scatter(msg0, dstv0, ssem0)

    plsc.subcore_barrier()
    pltpu.sync_copy(acc_sh.at[pl.ds(row0, RPT)],
                    out_hbm.at[pl.ds(cid * N2 + row0, RPT)])


def _sc_deg(dst, ew):
    k = pl.kernel(
        _sc_deg_body,
        out_type=jax.ShapeDtypeStruct((NC * N2, L), jnp.float32),
        mesh=_sc_mesh(),
        scratch_types=[
            pltpu.VMEM_SHARED((N2, L), jnp.float32),
            pltpu.VMEM((C,), jnp.int32),
            pltpu.VMEM((C,), jnp.int32),
            pltpu.VMEM((EPW,), jnp.float32),
            pltpu.VMEM((C, L), jnp.float32),
            pltpu.VMEM((C, L), jnp.float32),
            pltpu.VMEM((RPT, L), jnp.float32),
            pltpu.SemaphoreType.DMA,
            pltpu.SemaphoreType.DMA,
            pltpu.SemaphoreType.DMA,
            pltpu.SemaphoreType.DMA,
        ],
    )
    return k(dst, ew.reshape(NW, EPW))


def _sc_agg_body(g_hbm, src_hbm, dst_hbm, ew_hbm, out_hbm,
                 acc_sh, srcv0, srcv1, dstv0, dstv1, ew_all,
                 rows0, rows1, zbuf,
                 gsem0, gsem1, ssem0, ssem1, isem0, isem1):
    cid = lax.axis_index("c")
    sid = lax.axis_index("s")
    wid = sid * NC + cid
    base = wid * EPW
    row0 = sid * RPT

    lanes0 = jnp.zeros((L,), jnp.float32)

    def zb(j, _):
        for kk in range(D // L):
            zbuf[j, pl.ds(kk * L, L)] = lanes0
        return 0
    lax.fori_loop(0, ZR, zb, 0)
    for i in range(RPT // ZR):
        pltpu.sync_copy(zbuf, acc_sh.at[pl.ds(row0 + i * ZR, ZR)])

    pltpu.sync_copy(ew_hbm.at[wid], ew_all)
    plsc.subcore_barrier()

    def start_idx(c, srcv, dstv, sem):
        pltpu.async_copy(src_hbm.at[pl.ds(base + c * C, C)], srcv, sem)
        pltpu.async_copy(dst_hbm.at[pl.ds(base + c * C, C)], dstv, sem)

    def wait_idx(c, srcv, dstv, sem):
        pltpu.make_async_copy(src_hbm.at[pl.ds(base + c * C, C)], srcv,
                              sem).wait()
        pltpu.make_async_copy(dst_hbm.at[pl.ds(base + c * C, C)], dstv,
                              sem).wait()

    def start_gather(rows, srcv, sem):
        pltpu.async_copy(g_hbm.at[srcv], rows, sem)

    def wait_gather(rows, srcv, sem):
        pltpu.make_async_copy(g_hbm.at[srcv], rows, sem).wait()

    def scale(c, rows):
        def body(gi, _):
            j0 = gi * L
            ew16 = ew_all[pl.ds(c * C + j0, L)]
            for j in range(L):
                w = ew16[j]
                for kk in range(D // L):
                    rows[j0 + j, pl.ds(kk * L, L)] = (
                        rows[j0 + j, pl.ds(kk * L, L)] * w)
            return 0
        lax.fori_loop(0, C // L, body, 0)

    def start_scatter(rows, dstv, sem):
        pltpu.async_copy(rows, acc_sh.at[dstv], sem, add=True)

    def wait_scatter(rows, dstv, sem):
        pltpu.make_async_copy(rows, acc_sh.at[dstv], sem).wait()

    start_idx(0, srcv0, dstv0, isem0)
    start_idx(1, srcv1, dstv1, isem1)
    wait_idx(0, srcv0, dstv0, isem0)
    start_gather(rows0, srcv0, gsem0)
    wait_idx(1, srcv1, dstv1, isem1)
    start_gather(rows1, srcv1, gsem1)

    def pair(i, _):
        a = 2 * i
        b = a + 1
        wait_gather(rows0, srcv0, gsem0)
        scale(a, rows0)
        start_scatter(rows0, dstv0, ssem0)
        wait_gather(rows1, srcv1, gsem1)
        scale(b, rows1)
        start_scatter(rows1, dstv1, ssem1)

        wait_scatter(rows0, dstv0, ssem0)
        start_idx(a + 2, srcv0, dstv0, isem0)
        wait_idx(a + 2, srcv0, dstv0, isem0)
        start_gather(rows0, srcv0, gsem0)

        wait_scatter(rows1, dstv1, ssem1)

        @pl.when(b + 2 < NCHUNK)
        def _():
            start_idx(b + 2, srcv1, dstv1, isem1)
            wait_idx(b + 2, srcv1, dstv1, isem1)
            start_gather(rows1, srcv1, gsem1)
        return 0

    lax.fori_loop(0, NCHUNK // 2, pair, 0)

    last = NCHUNK - 1
    wait_gather(rows0, srcv0, gsem0)
    scale(last, rows0)
    start_scatter(rows0, dstv0, ssem0)
    wait_scatter(rows0, dstv0, ssem0)

    plsc.subcore_barrier()
    for i in range(RPT // ZR):
        pltpu.sync_copy(acc_sh.at[pl.ds(row0 + i * ZR, ZR)],
                        out_hbm.at[pl.ds(cid * N2 + row0 + i * ZR, ZR)])


def _sc_agg(g, src, dst, ew):
    k = pl.kernel(
        _sc_agg_body,
        out_type=jax.ShapeDtypeStruct((NC * N2, D), jnp.float32),
        mesh=_sc_mesh(),
        scratch_types=[
            pltpu.VMEM_SHARED((N2, D), jnp.float32),
            pltpu.VMEM((C,), jnp.int32),
            pltpu.VMEM((C,), jnp.int32),
            pltpu.VMEM((C,), jnp.int32),
            pltpu.VMEM((C,), jnp.int32),
            pltpu.VMEM((EPW,), jnp.float32),
            pltpu.VMEM((C, D), jnp.float32),
            pltpu.VMEM((C, D), jnp.float32),
            pltpu.VMEM((ZR, D), jnp.float32),
            pltpu.SemaphoreType.DMA,
            pltpu.SemaphoreType.DMA,
            pltpu.SemaphoreType.DMA,
            pltpu.SemaphoreType.DMA,
            pltpu.SemaphoreType.DMA,
            pltpu.SemaphoreType.DMA,
        ],
    )
    return k(g, src, dst, ew.reshape(NW, EPW))


BM = 256


def _tc1_body(d0_ref, d1_ref, x_ref, w_ref, g_ref, dinv_ref):
    s = jnp.sum(d0_ref[...] + d1_ref[...], axis=1, keepdims=True) + 1.0
    dv = lax.rsqrt(s)
    t = jnp.dot(x_ref[...], w_ref[...], preferred_element_type=jnp.float32)
    g_ref[...] = dv * t
    dinv_ref[...] = dv


def _tc1(degp0, degp1, x, W1):
    grid = (pl.cdiv(N, BM),)
    return pl.pallas_call(
        _tc1_body,
        grid=grid,
        in_specs=[
            pl.BlockSpec((BM, L), lambda i: (i, 0)),
            pl.BlockSpec((BM, L), lambda i: (i, 0)),
            pl.BlockSpec((BM, D), lambda i: (i, 0)),
            pl.BlockSpec((D, D), lambda i: (0, 0)),
        ],
        out_specs=[
            pl.BlockSpec((BM, D), lambda i: (i, 0)),
            pl.BlockSpec((BM, 1), lambda i: (i, 0)),
        ],
        out_shape=[
            jax.ShapeDtypeStruct((N, D), jnp.float32),
            jax.ShapeDtypeStruct((N, 1), jnp.float32),
        ],
    )(degp0, degp1, x, W1)


def _tc2_body(p0_ref, p1_ref, g_ref, dv_ref, b_ref, w_ref, g2_ref):
    dv = dv_ref[...]
    h = jnp.maximum(dv * (p0_ref[...] + p1_ref[...] + g_ref[...]) + b_ref[...],
                    0.0)
    t2 = jnp.dot(h, w_ref[...], preferred_element_type=jnp.float32)
    g2_ref[...] = dv * t2


def _tc2(p0, p1, g1, dinv, b1, W2):
    grid = (pl.cdiv(N, BM),)
    return pl.pallas_call(
        _tc2_body,
        grid=grid,
        in_specs=[
            pl.BlockSpec((BM, D), lambda i: (i, 0)),
            pl.BlockSpec((BM, D), lambda i: (i, 0)),
            pl.BlockSpec((BM, D), lambda i: (i, 0)),
            pl.BlockSpec((BM, 1), lambda i: (i, 0)),
            pl.BlockSpec((1, D), lambda i: (0, 0)),
            pl.BlockSpec((D, D), lambda i: (0, 0)),
        ],
        out_specs=pl.BlockSpec((BM, D), lambda i: (i, 0)),
        out_shape=jax.ShapeDtypeStruct((N, D), jnp.float32),
    )(p0, p1, g1, dinv, b1, W2)


def _tc3_body(p0_ref, p1_ref, g_ref, dv_ref, b_ref, out_ref):
    dv = dv_ref[...]
    out_ref[...] = jnp.maximum(
        dv * (p0_ref[...] + p1_ref[...] + g_ref[...]) + b_ref[...], 0.0)


def _tc3(p0, p1, g2, dinv, b2):
    grid = (pl.cdiv(N, BM),)
    return pl.pallas_call(
        _tc3_body,
        grid=grid,
        in_specs=[
            pl.BlockSpec((BM, D), lambda i: (i, 0)),
            pl.BlockSpec((BM, D), lambda i: (i, 0)),
            pl.BlockSpec((BM, D), lambda i: (i, 0)),
            pl.BlockSpec((BM, 1), lambda i: (i, 0)),
            pl.BlockSpec((1, D), lambda i: (0, 0)),
        ],
        out_specs=pl.BlockSpec((BM, D), lambda i: (i, 0)),
        out_shape=jax.ShapeDtypeStruct((N, D), jnp.float32),
    )(p0, p1, g2, dinv, b2)


@jax.jit
def _run(x, edge_index, edge_weight, W1, b1, W2, b2):
    src = edge_index[0]
    dst = edge_index[1]
    b1r = b1.reshape(1, D)
    b2r = b2.reshape(1, D)

    degp = _sc_deg(dst, edge_weight)
    g1, dinv = _tc1(degp[:N], degp[N2:N2 + N], x, W1)

    s1 = _sc_agg(g1, src, dst, edge_weight)
    g2 = _tc2(s1[:N], s1[N2:N2 + N], g1, dinv, b1r, W2)

    s2 = _sc_agg(g2, src, dst, edge_weight)
    h2 = _tc3(s2[:N], s2[N2:N2 + N], g2, dinv, b2r)
    return h2


def kernel(x, edge_index, edge_weight, W1, b1, W2, b2):
    return (_run(x, edge_index, edge_weight, W1, b1, W2, b2), None)


# speedup vs baseline: 20.0334x; 1.0873x over previous
import functools

import jax
import jax.numpy as jnp
from jax import lax
from jax.experimental import pallas as pl
from jax.experimental.pallas import tpu as pltpu
from jax.experimental.pallas import tpu_sc as plsc

N = 10000
E = 320000
D = 128

NC = 2
NS = 16
L = 16
NW = NC * NS
EPW = E // NW
C = 80
NCHUNK = EPW // C
N2 = 10240
RPT = N2 // NS
ZR = 32


def _sc_mesh():
    return plsc.VectorSubcoreMesh(core_axis_name="c", subcore_axis_name="s",
                                  num_cores=NC, num_subcores=NS)


def _sc_deg_body(dst_hbm, ew_hbm, out_hbm, acc_sh, ew_all,
                 dstv0, dstv1, msg0, msg1, zbuf,
                 ssem0, ssem1, isem0, isem1):
    cid = lax.axis_index("c")
    sid = lax.axis_index("s")
    wid = sid * NC + cid
    row0 = sid * RPT

    lanes0 = jnp.zeros((L,), jnp.float32)

    def zb(j, _):
        zbuf[j, :] = lanes0
        return 0
    lax.fori_loop(0, RPT, zb, 0)
    pltpu.sync_copy(zbuf, acc_sh.at[pl.ds(row0, RPT)])
    pltpu.sync_copy(ew_hbm.at[wid], ew_all)
    plsc.subcore_barrier()

    base = wid * EPW
    iota = lax.iota(jnp.int32, L)
    e0 = jnp.where(iota == 0, 1.0, 0.0).astype(jnp.float32)

    def start_idx(c, dstv, sem):
        pltpu.async_copy(dst_hbm.at[pl.ds(base + c * C, C)], dstv, sem)

    def wait_idx(c, dstv, sem):
        pltpu.make_async_copy(dst_hbm.at[pl.ds(base + c * C, C)], dstv,
                              sem).wait()

    def build(c, msg):
        def mrow(gi, _):
            j0 = gi * L
            ew16 = ew_all[pl.ds(c * C + j0, L)]
            for j in range(L):
                msg[j0 + j, :] = ew16[j] * e0
            return 0
        lax.fori_loop(0, C // L, mrow, 0)

    def start_scatter(msg, dstv, sem):
        pltpu.async_copy(msg, acc_sh.at[dstv], sem, add=True)

    def wait_scatter(msg, dstv, sem):
        pltpu.make_async_copy(msg, acc_sh.at[dstv], sem).wait()

    start_idx(0, dstv0, isem0)
    start_idx(1, dstv1, isem1)

    def pair(i, _):
        a = 2 * i
        b = a + 1
        build(a, msg0)
        wait_idx(a, dstv0, isem0)
        start_scatter(msg0, dstv0, ssem0)
        build(b, msg1)
        wait_idx(b, dstv1, isem1)
        start_scatter(msg1, dstv1, ssem1)
        wait_scatter(msg0, dstv0, ssem0)
        start_idx(a + 2, dstv0, isem0)
        wait_scatter(msg1, dstv1, ssem1)

        @pl.when(b + 2 < NCHUNK)
        def _():
            start_idx(b + 2, dstv1, isem1)
        return 0

    lax.fori_loop(0, NCHUNK // 2, pair, 0)
    last = NCHUNK - 1
    build(last, msg0)
    wait_idx(last, dstv0, isem0)
    start_scatter(msg0, dstv0, ssem0)
    wait_scatter(msg0, dstv0, ssem0)

    plsc.subcore_barrier()
    pltpu.sync_copy(acc_sh.at[pl.ds(row0, RPT)],
                    out_hbm.at[pl.ds(cid * N2 + row0, RPT)])


def _sc_deg(dst, ew):
    k = pl.kernel(
        _sc_deg_body,
        out_type=jax.ShapeDtypeStruct((NC * N2, L), jnp.float32),
        mesh=_sc_mesh(),
        scratch_types=[
            pltpu.VMEM_SHARED((N2, L), jnp.float32),
            pltpu.VMEM((EPW,), jnp.float32),
            pltpu.VMEM((C,), jnp.int32),
            pltpu.VMEM((C,), jnp.int32),
            pltpu.VMEM((C, L), jnp.float32),
            pltpu.VMEM((C, L), jnp.float32),
            pltpu.VMEM((RPT, L), jnp.float32),
            pltpu.SemaphoreType.DMA,
            pltpu.SemaphoreType.DMA,
            pltpu.SemaphoreType.DMA,
            pltpu.SemaphoreType.DMA,
        ],
    )
    return k(dst, ew.reshape(NW, EPW))


def _sc_agg_body(g_hbm, src_hbm, dst_hbm, ew_hbm, out_hbm,
                 acc_sh, dstall, ew_all,
                 srcv0, srcv1, dstv0, dstv1,
                 rows0, rows1, zbuf,
                 gsem0, gsem1, ssem0, ssem1, isem0, isem1):
    cid = lax.axis_index("c")
    sid = lax.axis_index("s")
    wid = sid * NC + cid
    base = wid * EPW
    row0 = sid * RPT

    lanes0 = jnp.zeros((L,), jnp.float32)

    def zb(j, _):
        for kk in range(D // L):
            zbuf[j, pl.ds(kk * L, L)] = lanes0
        return 0
    lax.fori_loop(0, ZR, zb, 0)
    for i in range(RPT // ZR):
        pltpu.async_copy(zbuf, acc_sh.at[pl.ds(row0 + i * ZR, ZR)], gsem0)

    pltpu.sync_copy(ew_hbm.at[wid], ew_all)
    pltpu.sync_copy(dst_hbm.at[wid], dstall)
    for i in range(RPT // ZR):
        pltpu.make_async_copy(zbuf, acc_sh.at[pl.ds(row0 + i * ZR, ZR)],
                              gsem0).wait()
    plsc.subcore_barrier()

    def start_src(c, srcv, sem):
        pltpu.async_copy(src_hbm.at[pl.ds(base + c * C, C)], srcv, sem)

    def wait_src(c, srcv, sem):
        pltpu.make_async_copy(src_hbm.at[pl.ds(base + c * C, C)], srcv,
                              sem).wait()

    def load_dst(c, dstv):
        for gi in range(C // L):
            dstv[pl.ds(gi * L, L)] = dstall[pl.ds(c * C + gi * L, L)]

    def start_gather(rows, srcv, sem):
        pltpu.async_copy(g_hbm.at[srcv], rows, sem)

    def wait_gather(rows, srcv, sem):
        pltpu.make_async_copy(g_hbm.at[srcv], rows, sem).wait()

    lane_sel = [jnp.full((L, 1), j, jnp.int32) for j in range(L)]
    dnums = lax.GatherDimensionNumbers(
        offset_dims=(), collapsed_slice_dims=(0,), start_index_map=(0,))

    def bcast(vec, j):
        return lax.gather(vec, lane_sel[j], dnums, (1,),
                          mode=lax.GatherScatterMode.PROMISE_IN_BOUNDS)

    def scale(c, rows):
        def body(gi, _):
            j0 = gi * L
            ew16 = ew_all[pl.ds(c * C + j0, L)]
            for j in range(L):
                w = bcast(ew16, j)
                for kk in range(D // L):
                    rows[j0 + j, pl.ds(kk * L, L)] = (
                        rows[j0 + j, pl.ds(kk * L, L)] * w)
            return 0
        lax.fori_loop(0, C // L, body, 0)

    def start_scatter(rows, dstv, sem):
        pltpu.async_copy(rows, acc_sh.at[dstv], sem, add=True)

    def wait_scatter(rows, dstv, sem):
        pltpu.make_async_copy(rows, acc_sh.at[dstv], sem).wait()

    start_src(0, srcv0, isem0)
    start_src(1, srcv1, isem1)
    wait_src(0, srcv0, isem0)
    start_gather(rows0, srcv0, gsem0)
    wait_src(1, srcv1, isem1)
    start_gather(rows1, srcv1, gsem1)

    def pair(i, _):
        a = 2 * i
        b = a + 1
        wait_gather(rows0, srcv0, gsem0)
        start_src(a + 2, srcv0, isem0)
        scale(a, rows0)
        load_dst(a, dstv0)
        start_scatter(rows0, dstv0, ssem0)

        wait_gather(rows1, srcv1, gsem1)

        @pl.when(b + 2 < NCHUNK)
        def _():
            start_src(b + 2, srcv1, isem1)
        scale(b, rows1)
        load_dst(b, dstv1)
        start_scatter(rows1, dstv1, ssem1)

        wait_scatter(rows0, dstv0, ssem0)
        wait_src(a + 2, srcv0, isem0)
        start_gather(rows0, srcv0, gsem0)

        wait_scatter(rows1, dstv1, ssem1)

        @pl.when(b + 2 < NCHUNK)
        def _():
            wait_src(b + 2, srcv1, isem1)
            start_gather(rows1, srcv1, gsem1)
        return 0

    lax.fori_loop(0, NCHUNK // 2, pair, 0)

    last = NCHUNK - 1
    wait_gather(rows0, srcv0, gsem0)
    scale(last, rows0)
    load_dst(last, dstv0)
    start_scatter(rows0, dstv0, ssem0)
    wait_scatter(rows0, dstv0, ssem0)

    plsc.subcore_barrier()
    pltpu.sync_copy(acc_sh.at[pl.ds(row0, RPT)],
                    out_hbm.at[pl.ds(cid * N2 + row0, RPT)])


def _sc_agg(g, src, dst, ew):
    k = pl.kernel(
        _sc_agg_body,
        out_type=jax.ShapeDtypeStruct((NC * N2, D), jnp.float32),
        mesh=_sc_mesh(),
        scratch_types=[
            pltpu.VMEM_SHARED((N2, D), jnp.float32),
            pltpu.VMEM((EPW,), jnp.int32),
            pltpu.VMEM((EPW,), jnp.float32),
            pltpu.VMEM((C,), jnp.int32),
            pltpu.VMEM((C,), jnp.int32),
            pltpu.VMEM((C,), jnp.int32),
            pltpu.VMEM((C,), jnp.int32),
            pltpu.VMEM((C, D), jnp.float32),
            pltpu.VMEM((C, D), jnp.float32),
            pltpu.VMEM((ZR, D), jnp.float32),
            pltpu.SemaphoreType.DMA,
            pltpu.SemaphoreType.DMA,
            pltpu.SemaphoreType.DMA,
            pltpu.SemaphoreType.DMA,
            pltpu.SemaphoreType.DMA,
            pltpu.SemaphoreType.DMA,
        ],
    )
    return k(g, src, dst.reshape(NW, EPW), ew.reshape(NW, EPW))


BM = 256


def _tc1_body(d0_ref, d1_ref, x_ref, w_ref, g_ref, dinv_ref):
    s = jnp.sum(d0_ref[...] + d1_ref[...], axis=1, keepdims=True) + 1.0
    dv = lax.rsqrt(s)
    t = jnp.dot(x_ref[...], w_ref[...], preferred_element_type=jnp.float32)
    g_ref[...] = dv * t
    dinv_ref[...] = dv


def _tc1(degp0, degp1, x, W1):
    grid = (pl.cdiv(N, BM),)
    return pl.pallas_call(
        _tc1_body,
        grid=grid,
        in_specs=[
            pl.BlockSpec((BM, L), lambda i: (i, 0)),
            pl.BlockSpec((BM, L), lambda i: (i, 0)),
            pl.BlockSpec((BM, D), lambda i: (i, 0)),
            pl.BlockSpec((D, D), lambda i: (0, 0)),
        ],
        out_specs=[
            pl.BlockSpec((BM, D), lambda i: (i, 0)),
            pl.BlockSpec((BM, 1), lambda i: (i, 0)),
        ],
        out_shape=[
            jax.ShapeDtypeStruct((N, D), jnp.float32),
            jax.ShapeDtypeStruct((N, 1), jnp.float32),
        ],
    )(degp0, degp1, x, W1)


def _tc2_body(p0_ref, p1_ref, g_ref, dv_ref, b_ref, w_ref, g2_ref):
    dv = dv_ref[...]
    h = jnp.maximum(dv * (p0_ref[...] + p1_ref[...] + g_ref[...]) + b_ref[...],
                    0.0)
    t2 = jnp.dot(h, w_ref[...], preferred_element_type=jnp.float32)
    g2_ref[...] = dv * t2


def _tc2(p0, p1, g1, dinv, b1, W2):
    grid = (pl.cdiv(N, BM),)
    return pl.pallas_call(
        _tc2_body,
        grid=grid,
        in_specs=[
            pl.BlockSpec((BM, D), lambda i: (i, 0)),
            pl.BlockSpec((BM, D), lambda i: (i, 0)),
            pl.BlockSpec((BM, D), lambda i: (i, 0)),
            pl.BlockSpec((BM, 1), lambda i: (i, 0)),
            pl.BlockSpec((1, D), lambda i: (0, 0)),
            pl.BlockSpec((D, D), lambda i: (0, 0)),
        ],
        out_specs=pl.BlockSpec((BM, D), lambda i: (i, 0)),
        out_shape=jax.ShapeDtypeStruct((N, D), jnp.float32),
    )(p0, p1, g1, dinv, b1, W2)


def _tc3_body(p0_ref, p1_ref, g_ref, dv_ref, b_ref, out_ref):
    dv = dv_ref[...]
    out_ref[...] = jnp.maximum(
        dv * (p0_ref[...] + p1_ref[...] + g_ref[...]) + b_ref[...], 0.0)


def _tc3(p0, p1, g2, dinv, b2):
    grid = (pl.cdiv(N, BM),)
    return pl.pallas_call(
        _tc3_body,
        grid=grid,
        in_specs=[
            pl.BlockSpec((BM, D), lambda i: (i, 0)),
            pl.BlockSpec((BM, D), lambda i: (i, 0)),
            pl.BlockSpec((BM, D), lambda i: (i, 0)),
            pl.BlockSpec((BM, 1), lambda i: (i, 0)),
            pl.BlockSpec((1, D), lambda i: (0, 0)),
        ],
        out_specs=pl.BlockSpec((BM, D), lambda i: (i, 0)),
        out_shape=jax.ShapeDtypeStruct((N, D), jnp.float32),
    )(p0, p1, g2, dinv, b2)


@jax.jit
def _run(x, edge_index, edge_weight, W1, b1, W2, b2):
    src = edge_index[0]
    dst = edge_index[1]
    b1r = b1.reshape(1, D)
    b2r = b2.reshape(1, D)

    degp = _sc_deg(dst, edge_weight)
    g1, dinv = _tc1(degp[:N], degp[N2:N2 + N], x, W1)

    s1 = _sc_agg(g1, src, dst, edge_weight)
    g2 = _tc2(s1[:N], s1[N2:N2 + N], g1, dinv, b1r, W2)

    s2 = _sc_agg(g2, src, dst, edge_weight)
    h2 = _tc3(s2[:N], s2[N2:N2 + N], g2, dinv, b2r)
    return h2


def kernel(x, edge_index, edge_weight, W1, b1, W2, b2):
    return (_run(x, edge_index, edge_weight, W1, b1, W2, b2), None)


# speedup vs baseline: 20.0586x; 1.0013x over previous
import functools

import jax
import jax.numpy as jnp
from jax import lax
from jax.experimental import pallas as pl
from jax.experimental.pallas import tpu as pltpu
from jax.experimental.pallas import tpu_sc as plsc

N = 10000
E = 320000
D = 128

NC = 2
NS = 16
L = 16
NW = NC * NS
EPW = E // NW
C = 80
NCHUNK = EPW // C
N2 = 10240
RPT = N2 // NS
ZR = 32


def _sc_mesh():
    return plsc.VectorSubcoreMesh(core_axis_name="c", subcore_axis_name="s",
                                  num_cores=NC, num_subcores=NS)


def _sc_deg_body(dst_hbm, ew_hbm, out_hbm, acc_sh, ew_all,
                 dstv0, dstv1, msg0, msg1, zbuf,
                 ssem0, ssem1, isem0, isem1):
    cid = lax.axis_index("c")
    sid = lax.axis_index("s")
    wid = sid * NC + cid
    row0 = sid * RPT

    lanes0 = jnp.zeros((L,), jnp.float32)

    def zb(j, _):
        zbuf[j, :] = lanes0
        return 0
    lax.fori_loop(0, RPT, zb, 0)
    pltpu.sync_copy(zbuf, acc_sh.at[pl.ds(row0, RPT)])
    pltpu.sync_copy(ew_hbm.at[wid], ew_all)
    plsc.subcore_barrier()

    base = wid * EPW
    iota = lax.iota(jnp.int32, L)
    e0 = jnp.where(iota == 0, 1.0, 0.0).astype(jnp.float32)

    def start_idx(c, dstv, sem):
        pltpu.async_copy(dst_hbm.at[pl.ds(base + c * C, C)], dstv, sem)

    def wait_idx(c, dstv, sem):
        pltpu.make_async_copy(dst_hbm.at[pl.ds(base + c * C, C)], dstv,
                              sem).wait()

    def build(c, msg):
        def mrow(gi, _):
            j0 = gi * L
            ew16 = ew_all[pl.ds(c * C + j0, L)]
            for j in range(L):
                msg[j0 + j, :] = ew16[j] * e0
            return 0
        lax.fori_loop(0, C // L, mrow, 0)

    def start_scatter(msg, dstv, sem):
        pltpu.async_copy(msg, acc_sh.at[dstv], sem, add=True)

    def wait_scatter(msg, dstv, sem):
        pltpu.make_async_copy(msg, acc_sh.at[dstv], sem).wait()

    start_idx(0, dstv0, isem0)
    start_idx(1, dstv1, isem1)

    def pair(i, _):
        a = 2 * i
        b = a + 1
        build(a, msg0)
        wait_idx(a, dstv0, isem0)
        start_scatter(msg0, dstv0, ssem0)
        build(b, msg1)
        wait_idx(b, dstv1, isem1)
        start_scatter(msg1, dstv1, ssem1)
        wait_scatter(msg0, dstv0, ssem0)
        start_idx(a + 2, dstv0, isem0)
        wait_scatter(msg1, dstv1, ssem1)

        @pl.when(b + 2 < NCHUNK)
        def _():
            start_idx(b + 2, dstv1, isem1)
        return 0

    lax.fori_loop(0, NCHUNK // 2, pair, 0)
    last = NCHUNK - 1
    build(last, msg0)
    wait_idx(last, dstv0, isem0)
    start_scatter(msg0, dstv0, ssem0)
    wait_scatter(msg0, dstv0, ssem0)

    plsc.subcore_barrier()
    pltpu.sync_copy(acc_sh.at[pl.ds(row0, RPT)],
                    out_hbm.at[pl.ds(cid * N2 + row0, RPT)])


def _sc_deg(dst, ew):
    k = pl.kernel(
        _sc_deg_body,
        out_type=jax.ShapeDtypeStruct((NC * N2, L), jnp.float32),
        mesh=_sc_mesh(),
        scratch_types=[
            pltpu.VMEM_SHARED((N2, L), jnp.float32),
            pltpu.VMEM((EPW,), jnp.float32),
            pltpu.VMEM((C,), jnp.int32),
            pltpu.VMEM((C,), jnp.int32),
            pltpu.VMEM((C, L), jnp.float32),
            pltpu.VMEM((C, L), jnp.float32),
            pltpu.VMEM((RPT, L), jnp.float32),
            pltpu.SemaphoreType.DMA,
            pltpu.SemaphoreType.DMA,
            pltpu.SemaphoreType.DMA,
            pltpu.SemaphoreType.DMA,
        ],
    )
    return k(dst, ew.reshape(NW, EPW))


def _sc_agg_body(g_hbm, src_hbm, dst_hbm, ew_hbm, out_hbm,
                 acc_sh, dstall, ew_all,
                 srcv0, srcv1, dstv0, dstv1,
                 rows0, rows1, zbuf,
                 gsem0, gsem1, ssem0, ssem1, isem0, isem1):
    cid = lax.axis_index("c")
    sid = lax.axis_index("s")
    wid = sid * NC + cid
    base = wid * EPW
    row0 = sid * RPT

    lanes0 = jnp.zeros((L,), jnp.float32)

    def zb(j, _):
        for kk in range(D // L):
            zbuf[j, pl.ds(kk * L, L)] = lanes0
        return 0
    lax.fori_loop(0, ZR, zb, 0)
    for i in range(RPT // ZR):
        pltpu.async_copy(zbuf, acc_sh.at[pl.ds(row0 + i * ZR, ZR)], gsem0)

    pltpu.sync_copy(ew_hbm.at[wid], ew_all)
    pltpu.sync_copy(dst_hbm.at[wid], dstall)
    for i in range(RPT // ZR):
        pltpu.make_async_copy(zbuf, acc_sh.at[pl.ds(row0 + i * ZR, ZR)],
                              gsem0).wait()
    plsc.subcore_barrier()

    def start_src(c, srcv, sem):
        pltpu.async_copy(src_hbm.at[pl.ds(base + c * C, C)], srcv, sem)

    def wait_src(c, srcv, sem):
        pltpu.make_async_copy(src_hbm.at[pl.ds(base + c * C, C)], srcv,
                              sem).wait()

    def load_dst(c, dstv):
        for gi in range(C // L):
            dstv[pl.ds(gi * L, L)] = dstall[pl.ds(c * C + gi * L, L)]

    def start_gather(rows, srcv, sem):
        pltpu.async_copy(g_hbm.at[srcv], rows, sem)

    def wait_gather(rows, srcv, sem):
        pltpu.make_async_copy(g_hbm.at[srcv], rows, sem).wait()

    lane_sel = [jnp.full((L, 1), j, jnp.int32) for j in range(L)]
    dnums = lax.GatherDimensionNumbers(
        offset_dims=(), collapsed_slice_dims=(0,), start_index_map=(0,))

    def bcast(vec, j):
        return lax.gather(vec, lane_sel[j], dnums, (1,),
                          mode=lax.GatherScatterMode.PROMISE_IN_BOUNDS)

    def scale(c, rows):
        def body(gi, _):
            j0 = gi * L
            ew16 = ew_all[pl.ds(c * C + j0, L)]
            for j in range(L):
                w = ew16[j]
                for kk in range(D // L):
                    rows[j0 + j, pl.ds(kk * L, L)] = (
                        rows[j0 + j, pl.ds(kk * L, L)] * w)
            return 0
        lax.fori_loop(0, C // L, body, 0)

    def start_scatter(rows, dstv, sem):
        pltpu.async_copy(rows, acc_sh.at[dstv], sem, add=True)

    def wait_scatter(rows, dstv, sem):
        pltpu.make_async_copy(rows, acc_sh.at[dstv], sem).wait()

    start_src(0, srcv0, isem0)
    start_src(1, srcv1, isem1)
    wait_src(0, srcv0, isem0)
    start_gather(rows0, srcv0, gsem0)
    wait_src(1, srcv1, isem1)
    start_gather(rows1, srcv1, gsem1)

    def pair(i, _):
        a = 2 * i
        b = a + 1
        wait_gather(rows0, srcv0, gsem0)
        start_src(a + 2, srcv0, isem0)
        scale(a, rows0)
        load_dst(a, dstv0)
        start_scatter(rows0, dstv0, ssem0)

        wait_gather(rows1, srcv1, gsem1)

        @pl.when(b + 2 < NCHUNK)
        def _():
            start_src(b + 2, srcv1, isem1)
        scale(b, rows1)
        load_dst(b, dstv1)
        start_scatter(rows1, dstv1, ssem1)

        wait_scatter(rows0, dstv0, ssem0)
        wait_src(a + 2, srcv0, isem0)
        start_gather(rows0, srcv0, gsem0)

        wait_scatter(rows1, dstv1, ssem1)

        @pl.when(b + 2 < NCHUNK)
        def _():
            wait_src(b + 2, srcv1, isem1)
            start_gather(rows1, srcv1, gsem1)
        return 0

    lax.fori_loop(0, NCHUNK // 2, pair, 0)

    last = NCHUNK - 1
    wait_gather(rows0, srcv0, gsem0)
    scale(last, rows0)
    load_dst(last, dstv0)
    start_scatter(rows0, dstv0, ssem0)
    wait_scatter(rows0, dstv0, ssem0)

    plsc.subcore_barrier()
    pltpu.sync_copy(acc_sh.at[pl.ds(row0, RPT)],
                    out_hbm.at[pl.ds(cid * N2 + row0, RPT)])


def _sc_agg(g, src, dst, ew):
    k = pl.kernel(
        _sc_agg_body,
        out_type=jax.ShapeDtypeStruct((NC * N2, D), jnp.float32),
        mesh=_sc_mesh(),
        scratch_types=[
            pltpu.VMEM_SHARED((N2, D), jnp.float32),
            pltpu.VMEM((EPW,), jnp.int32),
            pltpu.VMEM((EPW,), jnp.float32),
            pltpu.VMEM((C,), jnp.int32),
            pltpu.VMEM((C,), jnp.int32),
            pltpu.VMEM((C,), jnp.int32),
            pltpu.VMEM((C,), jnp.int32),
            pltpu.VMEM((C, D), jnp.float32),
            pltpu.VMEM((C, D), jnp.float32),
            pltpu.VMEM((ZR, D), jnp.float32),
            pltpu.SemaphoreType.DMA,
            pltpu.SemaphoreType.DMA,
            pltpu.SemaphoreType.DMA,
            pltpu.SemaphoreType.DMA,
            pltpu.SemaphoreType.DMA,
            pltpu.SemaphoreType.DMA,
        ],
    )
    return k(g, src, dst.reshape(NW, EPW), ew.reshape(NW, EPW))


BM = 256


def _tc1_body(d0_ref, d1_ref, x_ref, w_ref, g_ref, dinv_ref):
    s = jnp.sum(d0_ref[...] + d1_ref[...], axis=1, keepdims=True) + 1.0
    dv = lax.rsqrt(s)
    t = jnp.dot(x_ref[...], w_ref[...], preferred_element_type=jnp.float32)
    g_ref[...] = dv * t
    dinv_ref[...] = dv


def _tc1(degp0, degp1, x, W1):
    grid = (pl.cdiv(N, BM),)
    return pl.pallas_call(
        _tc1_body,
        grid=grid,
        in_specs=[
            pl.BlockSpec((BM, L), lambda i: (i, 0)),
            pl.BlockSpec((BM, L), lambda i: (i, 0)),
            pl.BlockSpec((BM, D), lambda i: (i, 0)),
            pl.BlockSpec((D, D), lambda i: (0, 0)),
        ],
        out_specs=[
            pl.BlockSpec((BM, D), lambda i: (i, 0)),
            pl.BlockSpec((BM, 1), lambda i: (i, 0)),
        ],
        out_shape=[
            jax.ShapeDtypeStruct((N, D), jnp.float32),
            jax.ShapeDtypeStruct((N, 1), jnp.float32),
        ],
    )(degp0, degp1, x, W1)


def _tc2_body(p0_ref, p1_ref, g_ref, dv_ref, b_ref, w_ref, g2_ref):
    dv = dv_ref[...]
    h = jnp.maximum(dv * (p0_ref[...] + p1_ref[...] + g_ref[...]) + b_ref[...],
                    0.0)
    t2 = jnp.dot(h, w_ref[...], preferred_element_type=jnp.float32)
    g2_ref[...] = dv * t2


def _tc2(p0, p1, g1, dinv, b1, W2):
    grid = (pl.cdiv(N, BM),)
    return pl.pallas_call(
        _tc2_body,
        grid=grid,
        in_specs=[
            pl.BlockSpec((BM, D), lambda i: (i, 0)),
            pl.BlockSpec((BM, D), lambda i: (i, 0)),
            pl.BlockSpec((BM, D), lambda i: (i, 0)),
            pl.BlockSpec((BM, 1), lambda i: (i, 0)),
            pl.BlockSpec((1, D), lambda i: (0, 0)),
            pl.BlockSpec((D, D), lambda i: (0, 0)),
        ],
        out_specs=pl.BlockSpec((BM, D), lambda i: (i, 0)),
        out_shape=jax.ShapeDtypeStruct((N, D), jnp.float32),
    )(p0, p1, g1, dinv, b1, W2)


def _tc3_body(p0_ref, p1_ref, g_ref, dv_ref, b_ref, out_ref):
    dv = dv_ref[...]
    out_ref[...] = jnp.maximum(
        dv * (p0_ref[...] + p1_ref[...] + g_ref[...]) + b_ref[...], 0.0)


def _tc3(p0, p1, g2, dinv, b2):
    grid = (pl.cdiv(N, BM),)
    return pl.pallas_call(
        _tc3_body,
        grid=grid,
        in_specs=[
            pl.BlockSpec((BM, D), lambda i: (i, 0)),
            pl.BlockSpec((BM, D), lambda i: (i, 0)),
            pl.BlockSpec((BM, D), lambda i: (i, 0)),
            pl.BlockSpec((BM, 1), lambda i: (i, 0)),
            pl.BlockSpec((1, D), lambda i: (0, 0)),
        ],
        out_specs=pl.BlockSpec((BM, D), lambda i: (i, 0)),
        out_shape=jax.ShapeDtypeStruct((N, D), jnp.float32),
    )(p0, p1, g2, dinv, b2)


@jax.jit
def _run(x, edge_index, edge_weight, W1, b1, W2, b2):
    src = edge_index[0]
    dst = edge_index[1]
    b1r = b1.reshape(1, D)
    b2r = b2.reshape(1, D)

    degp = _sc_deg(dst, edge_weight)
    g1, dinv = _tc1(degp[:N], degp[N2:N2 + N], x, W1)

    s1 = _sc_agg(g1, src, dst, edge_weight)
    g2 = _tc2(s1[:N], s1[N2:N2 + N], g1, dinv, b1r, W2)

    s2 = _sc_agg(g2, src, dst, edge_weight)
    h2 = _tc3(s2[:N], s2[N2:N2 + N], g2, dinv, b2r)
    return h2


def kernel(x, edge_index, edge_weight, W1, b1, W2, b2):
    return (_run(x, edge_index, edge_weight, W1, b1, W2, b2), None)


# speedup vs baseline: 22.1090x; 1.1022x over previous
import functools

import jax
import jax.numpy as jnp
from jax import lax
from jax.experimental import pallas as pl
from jax.experimental.pallas import tpu as pltpu
from jax.experimental.pallas import tpu_sc as plsc

N = 10000
E = 320000
D = 128

NC = 2
NS = 16
L = 16
NW = NC * NS
EPW = E // NW
C = 80
NCHUNK = EPW // C
N2 = 10240
RPT = N2 // NS
ZR = 32


def _sc_mesh():
    return plsc.VectorSubcoreMesh(core_axis_name="c", subcore_axis_name="s",
                                  num_cores=NC, num_subcores=NS)


def _sc_deg_body(dst_hbm, ew_hbm, out_hbm, acc_sh, ew_all,
                 dstv0, dstv1, msg0, msg1, zbuf,
                 ssem0, ssem1, isem0, isem1):
    cid = lax.axis_index("c")
    sid = lax.axis_index("s")
    wid = sid * NC + cid
    row0 = sid * RPT

    lanes0 = jnp.zeros((L,), jnp.float32)

    def zb(j, _):
        zbuf[j, :] = lanes0
        return 0
    lax.fori_loop(0, RPT, zb, 0)
    pltpu.sync_copy(zbuf, acc_sh.at[pl.ds(row0, RPT)])
    pltpu.sync_copy(ew_hbm.at[wid], ew_all)
    plsc.subcore_barrier()

    base = wid * EPW
    iota = lax.iota(jnp.int32, L)
    e0 = jnp.where(iota == 0, 1.0, 0.0).astype(jnp.float32)

    def start_idx(c, dstv, sem):
        pltpu.async_copy(dst_hbm.at[pl.ds(base + c * C, C)], dstv, sem)

    def wait_idx(c, dstv, sem):
        pltpu.make_async_copy(dst_hbm.at[pl.ds(base + c * C, C)], dstv,
                              sem).wait()

    def build(c, msg):
        def mrow(gi, _):
            j0 = gi * L
            ew16 = ew_all[pl.ds(c * C + j0, L)]
            for j in range(L):
                msg[j0 + j, :] = ew16[j] * e0
            return 0
        lax.fori_loop(0, C // L, mrow, 0)

    def start_scatter(msg, dstv, sem):
        pltpu.async_copy(msg, acc_sh.at[dstv], sem, add=True)

    def wait_scatter(msg, dstv, sem):
        pltpu.make_async_copy(msg, acc_sh.at[dstv], sem).wait()

    start_idx(0, dstv0, isem0)
    start_idx(1, dstv1, isem1)

    def pair(i, _):
        a = 2 * i
        b = a + 1
        build(a, msg0)
        wait_idx(a, dstv0, isem0)
        start_scatter(msg0, dstv0, ssem0)
        build(b, msg1)
        wait_idx(b, dstv1, isem1)
        start_scatter(msg1, dstv1, ssem1)
        wait_scatter(msg0, dstv0, ssem0)

        @pl.when(a + 2 < NCHUNK)
        def _():
            start_idx(a + 2, dstv0, isem0)
        wait_scatter(msg1, dstv1, ssem1)

        @pl.when(b + 2 < NCHUNK)
        def _():
            start_idx(b + 2, dstv1, isem1)
        return 0

    lax.fori_loop(0, NCHUNK // 2, pair, 0)
    if NCHUNK % 2 == 1:
        last = NCHUNK - 1
        build(last, msg0)
        wait_idx(last, dstv0, isem0)
        start_scatter(msg0, dstv0, ssem0)
        wait_scatter(msg0, dstv0, ssem0)

    plsc.subcore_barrier()
    pltpu.sync_copy(acc_sh.at[pl.ds(row0, RPT)],
                    out_hbm.at[pl.ds(cid * N2 + row0, RPT)])


def _sc_deg(dst, ew):
    k = pl.kernel(
        _sc_deg_body,
        out_type=jax.ShapeDtypeStruct((NC * N2, L), jnp.float32),
        mesh=_sc_mesh(),
        scratch_types=[
            pltpu.VMEM_SHARED((N2, L), jnp.float32),
            pltpu.VMEM((EPW,), jnp.float32),
            pltpu.VMEM((C,), jnp.int32),
            pltpu.VMEM((C,), jnp.int32),
            pltpu.VMEM((C, L), jnp.float32),
            pltpu.VMEM((C, L), jnp.float32),
            pltpu.VMEM((RPT, L), jnp.float32),
            pltpu.SemaphoreType.DMA,
            pltpu.SemaphoreType.DMA,
            pltpu.SemaphoreType.DMA,
            pltpu.SemaphoreType.DMA,
        ],
    )
    return k(dst, ew.reshape(NW, EPW))


def _sc_agg_body(g_hbm, src_hbm, dst_hbm, ew_hbm, out_hbm,
                 acc_sh, dstall, ew_all,
                 srcv0, srcv1, dstv0, dstv1,
                 rows0, rows1, zbuf,
                 gsem0, gsem1, ssem0, ssem1, isem0, isem1):
    cid = lax.axis_index("c")
    sid = lax.axis_index("s")
    wid = sid * NC + cid
    base = wid * EPW
    row0 = sid * RPT

    lanes0 = jnp.zeros((L,), jnp.float32)

    def zb(j, _):
        for kk in range(D // L):
            zbuf[j, pl.ds(kk * L, L)] = lanes0
        return 0
    lax.fori_loop(0, ZR, zb, 0)
    for i in range(RPT // ZR):
        pltpu.async_copy(zbuf, acc_sh.at[pl.ds(row0 + i * ZR, ZR)], gsem0)

    pltpu.sync_copy(ew_hbm.at[wid], ew_all)
    pltpu.sync_copy(dst_hbm.at[wid], dstall)
    for i in range(RPT // ZR):
        pltpu.make_async_copy(zbuf, acc_sh.at[pl.ds(row0 + i * ZR, ZR)],
                              gsem0).wait()
    plsc.subcore_barrier()

    def start_src(c, srcv, sem):
        pltpu.async_copy(src_hbm.at[pl.ds(base + c * C, C)], srcv, sem)

    def wait_src(c, srcv, sem):
        pltpu.make_async_copy(src_hbm.at[pl.ds(base + c * C, C)], srcv,
                              sem).wait()

    def load_dst(c, dstv):
        for gi in range(C // L):
            dstv[pl.ds(gi * L, L)] = dstall[pl.ds(c * C + gi * L, L)]

    def start_gather(rows, srcv, sem):
        pltpu.async_copy(g_hbm.at[srcv], rows, sem)

    def wait_gather(rows, srcv, sem):
        pltpu.make_async_copy(g_hbm.at[srcv], rows, sem).wait()

    lane_sel = [jnp.full((L, 1), j, jnp.int32) for j in range(L)]
    dnums = lax.GatherDimensionNumbers(
        offset_dims=(), collapsed_slice_dims=(0,), start_index_map=(0,))

    def bcast(vec, j):
        return lax.gather(vec, lane_sel[j], dnums, (1,),
                          mode=lax.GatherScatterMode.PROMISE_IN_BOUNDS)

    def scale(c, rows):
        def body(gi, _):
            j0 = gi * L
            ew16 = ew_all[pl.ds(c * C + j0, L)]
            for j in range(L):
                w = ew16[j]
                for kk in range(D // L):
                    rows[j0 + j, pl.ds(kk * L, L)] = (
                        rows[j0 + j, pl.ds(kk * L, L)] * w)
            return 0
        lax.fori_loop(0, C // L, body, 0)

    def start_scatter(rows, dstv, sem):
        pltpu.async_copy(rows, acc_sh.at[dstv], sem, add=True)

    def wait_scatter(rows, dstv, sem):
        pltpu.make_async_copy(rows, acc_sh.at[dstv], sem).wait()

    start_src(0, srcv0, isem0)
    start_src(1, srcv1, isem1)
    wait_src(0, srcv0, isem0)
    start_gather(rows0, srcv0, gsem0)
    wait_src(1, srcv1, isem1)
    start_gather(rows1, srcv1, gsem1)

    def pair(i, _):
        a = 2 * i
        b = a + 1
        wait_gather(rows0, srcv0, gsem0)

        @pl.when(a + 2 < NCHUNK)
        def _():
            start_src(a + 2, srcv0, isem0)
        scale(a, rows0)
        load_dst(a, dstv0)
        start_scatter(rows0, dstv0, ssem0)

        wait_gather(rows1, srcv1, gsem1)

        @pl.when(b + 2 < NCHUNK)
        def _():
            start_src(b + 2, srcv1, isem1)
        scale(b, rows1)
        load_dst(b, dstv1)
        start_scatter(rows1, dstv1, ssem1)

        wait_scatter(rows0, dstv0, ssem0)

        @pl.when(a + 2 < NCHUNK)
        def _():
            wait_src(a + 2, srcv0, isem0)
            start_gather(rows0, srcv0, gsem0)

        wait_scatter(rows1, dstv1, ssem1)

        @pl.when(b + 2 < NCHUNK)
        def _():
            wait_src(b + 2, srcv1, isem1)
            start_gather(rows1, srcv1, gsem1)
        return 0

    lax.fori_loop(0, NCHUNK // 2, pair, 0)

    if NCHUNK % 2 == 1:
        last = NCHUNK - 1
        wait_gather(rows0, srcv0, gsem0)
        scale(last, rows0)
        load_dst(last, dstv0)
        start_scatter(rows0, dstv0, ssem0)
        wait_scatter(rows0, dstv0, ssem0)

    plsc.subcore_barrier()
    pltpu.sync_copy(acc_sh.at[pl.ds(row0, RPT)],
                    out_hbm.at[pl.ds(cid * N2 + row0, RPT)])


def _sc_agg(g, src, dst, ew):
    k = pl.kernel(
        _sc_agg_body,
        out_type=jax.ShapeDtypeStruct((NC * N2, D), jnp.float32),
        mesh=_sc_mesh(),
        scratch_types=[
            pltpu.VMEM_SHARED((N2, D), jnp.float32),
            pltpu.VMEM((EPW,), jnp.int32),
            pltpu.VMEM((EPW,), jnp.float32),
            pltpu.VMEM((C,), jnp.int32),
            pltpu.VMEM((C,), jnp.int32),
            pltpu.VMEM((C,), jnp.int32),
            pltpu.VMEM((C,), jnp.int32),
            pltpu.VMEM((C, D), jnp.float32),
            pltpu.VMEM((C, D), jnp.float32),
            pltpu.VMEM((ZR, D), jnp.float32),
            pltpu.SemaphoreType.DMA,
            pltpu.SemaphoreType.DMA,
            pltpu.SemaphoreType.DMA,
            pltpu.SemaphoreType.DMA,
            pltpu.SemaphoreType.DMA,
            pltpu.SemaphoreType.DMA,
        ],
    )
    return k(g, src, dst.reshape(NW, EPW), ew.reshape(NW, EPW))


BM = 1024


def _tc1_body(d0_ref, d1_ref, x_ref, w_ref, g_ref, dinv_ref):
    s = jnp.sum(d0_ref[...] + d1_ref[...], axis=1, keepdims=True) + 1.0
    dv = lax.rsqrt(s)
    t = jnp.dot(x_ref[...], w_ref[...], preferred_element_type=jnp.float32)
    g_ref[...] = dv * t
    dinv_ref[...] = dv


def _tc1(degp0, degp1, x, W1):
    grid = (pl.cdiv(N, BM),)
    return pl.pallas_call(
        _tc1_body,
        grid=grid,
        in_specs=[
            pl.BlockSpec((BM, L), lambda i: (i, 0)),
            pl.BlockSpec((BM, L), lambda i: (i, 0)),
            pl.BlockSpec((BM, D), lambda i: (i, 0)),
            pl.BlockSpec((D, D), lambda i: (0, 0)),
        ],
        out_specs=[
            pl.BlockSpec((BM, D), lambda i: (i, 0)),
            pl.BlockSpec((BM, 1), lambda i: (i, 0)),
        ],
        out_shape=[
            jax.ShapeDtypeStruct((N, D), jnp.float32),
            jax.ShapeDtypeStruct((N, 1), jnp.float32),
        ],
    )(degp0, degp1, x, W1)


def _tc2_body(p0_ref, p1_ref, g_ref, dv_ref, b_ref, w_ref, g2_ref):
    dv = dv_ref[...]
    h = jnp.maximum(dv * (p0_ref[...] + p1_ref[...] + g_ref[...]) + b_ref[...],
                    0.0)
    t2 = jnp.dot(h, w_ref[...], preferred_element_type=jnp.float32)
    g2_ref[...] = dv * t2


def _tc2(p0, p1, g1, dinv, b1, W2):
    grid = (pl.cdiv(N, BM),)
    return pl.pallas_call(
        _tc2_body,
        grid=grid,
        in_specs=[
            pl.BlockSpec((BM, D), lambda i: (i, 0)),
            pl.BlockSpec((BM, D), lambda i: (i, 0)),
            pl.BlockSpec((BM, D), lambda i: (i, 0)),
            pl.BlockSpec((BM, 1), lambda i: (i, 0)),
            pl.BlockSpec((1, D), lambda i: (0, 0)),
            pl.BlockSpec((D, D), lambda i: (0, 0)),
        ],
        out_specs=pl.BlockSpec((BM, D), lambda i: (i, 0)),
        out_shape=jax.ShapeDtypeStruct((N, D), jnp.float32),
    )(p0, p1, g1, dinv, b1, W2)


def _tc3_body(p0_ref, p1_ref, g_ref, dv_ref, b_ref, out_ref):
    dv = dv_ref[...]
    out_ref[...] = jnp.maximum(
        dv * (p0_ref[...] + p1_ref[...] + g_ref[...]) + b_ref[...], 0.0)


def _tc3(p0, p1, g2, dinv, b2):
    grid = (pl.cdiv(N, BM),)
    return pl.pallas_call(
        _tc3_body,
        grid=grid,
        in_specs=[
            pl.BlockSpec((BM, D), lambda i: (i, 0)),
            pl.BlockSpec((BM, D), lambda i: (i, 0)),
            pl.BlockSpec((BM, D), lambda i: (i, 0)),
            pl.BlockSpec((BM, 1), lambda i: (i, 0)),
            pl.BlockSpec((1, D), lambda i: (0, 0)),
        ],
        out_specs=pl.BlockSpec((BM, D), lambda i: (i, 0)),
        out_shape=jax.ShapeDtypeStruct((N, D), jnp.float32),
    )(p0, p1, g2, dinv, b2)


@jax.jit
def _run(x, edge_index, edge_weight, W1, b1, W2, b2):
    src = edge_index[0]
    dst = edge_index[1]
    b1r = b1.reshape(1, D)
    b2r = b2.reshape(1, D)

    degp = _sc_deg(dst, edge_weight)
    g1, dinv = _tc1(degp[:N], degp[N2:N2 + N], x, W1)

    s1 = _sc_agg(g1, src, dst, edge_weight)
    g2 = _tc2(s1[:N], s1[N2:N2 + N], g1, dinv, b1r, W2)

    s2 = _sc_agg(g2, src, dst, edge_weight)
    h2 = _tc3(s2[:N], s2[N2:N2 + N], g2, dinv, b2r)
    return h2


def kernel(x, edge_index, edge_weight, W1, b1, W2, b2):
    return (_run(x, edge_index, edge_weight, W1, b1, W2, b2), None)


# speedup vs baseline: 22.4275x; 1.0144x over previous
import functools

import jax
import jax.numpy as jnp
from jax import lax
from jax.experimental import pallas as pl
from jax.experimental.pallas import tpu as pltpu
from jax.experimental.pallas import tpu_sc as plsc

N = 10000
E = 320000
D = 128

NC = 2
NS = 16
L = 16
NW = NC * NS
EPW = E // NW
C = 80
NCHUNK = EPW // C
N2 = 10240
RPT = N2 // NS
ZR = 32


def _sc_mesh():
    return plsc.VectorSubcoreMesh(core_axis_name="c", subcore_axis_name="s",
                                  num_cores=NC, num_subcores=NS)


def _sc_deg_body(dst_hbm, ew_hbm, out_hbm, acc_sh, ew_all,
                 dstv0, dstv1, msg0, msg1, zbuf,
                 ssem0, ssem1, isem0, isem1):
    cid = lax.axis_index("c")
    sid = lax.axis_index("s")
    wid = sid * NC + cid
    row0 = sid * RPT

    lanes0 = jnp.zeros((L,), jnp.float32)

    def zb(j, _):
        zbuf[j, :] = lanes0
        return 0
    lax.fori_loop(0, RPT, zb, 0)
    pltpu.sync_copy(zbuf, acc_sh.at[pl.ds(row0, RPT)])
    pltpu.sync_copy(ew_hbm.at[wid], ew_all)
    plsc.subcore_barrier()

    base = wid * EPW
    iota = lax.iota(jnp.int32, L)
    e0 = jnp.where(iota == 0, 1.0, 0.0).astype(jnp.float32)

    def start_idx(c, dstv, sem):
        pltpu.async_copy(dst_hbm.at[pl.ds(base + c * C, C)], dstv, sem)

    def wait_idx(c, dstv, sem):
        pltpu.make_async_copy(dst_hbm.at[pl.ds(base + c * C, C)], dstv,
                              sem).wait()

    def build(c, msg):
        def mrow(gi, _):
            j0 = gi * L
            ew16 = ew_all[pl.ds(c * C + j0, L)]
            for j in range(L):
                msg[j0 + j, :] = ew16[j] * e0
            return 0
        lax.fori_loop(0, C // L, mrow, 0)

    def start_scatter(msg, dstv, sem):
        pltpu.async_copy(msg, acc_sh.at[dstv], sem, add=True)

    def wait_scatter(msg, dstv, sem):
        pltpu.make_async_copy(msg, acc_sh.at[dstv], sem).wait()

    start_idx(0, dstv0, isem0)
    start_idx(1, dstv1, isem1)

    def pair(i, _):
        a = 2 * i
        b = a + 1
        build(a, msg0)
        wait_idx(a, dstv0, isem0)
        start_scatter(msg0, dstv0, ssem0)
        build(b, msg1)
        wait_idx(b, dstv1, isem1)
        start_scatter(msg1, dstv1, ssem1)
        wait_scatter(msg0, dstv0, ssem0)

        @pl.when(a + 2 < NCHUNK)
        def _():
            start_idx(a + 2, dstv0, isem0)
        wait_scatter(msg1, dstv1, ssem1)

        @pl.when(b + 2 < NCHUNK)
        def _():
            start_idx(b + 2, dstv1, isem1)
        return 0

    lax.fori_loop(0, NCHUNK // 2, pair, 0)
    if NCHUNK % 2 == 1:
        last = NCHUNK - 1
        build(last, msg0)
        wait_idx(last, dstv0, isem0)
        start_scatter(msg0, dstv0, ssem0)
        wait_scatter(msg0, dstv0, ssem0)

    plsc.subcore_barrier()
    pltpu.sync_copy(acc_sh.at[pl.ds(row0, RPT)],
                    out_hbm.at[pl.ds(cid * N2 + row0, RPT)])


def _sc_deg(dst, ew):
    k = pl.kernel(
        _sc_deg_body,
        out_type=jax.ShapeDtypeStruct((NC * N2, L), jnp.float32),
        mesh=_sc_mesh(),
        scratch_types=[
            pltpu.VMEM_SHARED((N2, L), jnp.float32),
            pltpu.VMEM((EPW,), jnp.float32),
            pltpu.VMEM((C,), jnp.int32),
            pltpu.VMEM((C,), jnp.int32),
            pltpu.VMEM((C, L), jnp.float32),
            pltpu.VMEM((C, L), jnp.float32),
            pltpu.VMEM((RPT, L), jnp.float32),
            pltpu.SemaphoreType.DMA,
            pltpu.SemaphoreType.DMA,
            pltpu.SemaphoreType.DMA,
            pltpu.SemaphoreType.DMA,
        ],
    )
    return k(dst, ew.reshape(NW, EPW))


def _sc_agg_body(g_hbm, src_hbm, dst_hbm, ew_hbm, out_hbm,
                 acc_sh, dstall, ew_all,
                 srcv0, srcv1, dstv0, dstv1,
                 rows0, rows1, zbuf,
                 gsem0, gsem1, ssem0, ssem1, isem0, isem1):
    cid = lax.axis_index("c")
    sid = lax.axis_index("s")
    wid = sid * NC + cid
    base = wid * EPW
    row0 = sid * RPT

    lanes0 = jnp.zeros((L,), jnp.float32)

    def zb(j, _):
        for kk in range(D // L):
            zbuf[j, pl.ds(kk * L, L)] = lanes0
        return 0
    lax.fori_loop(0, ZR, zb, 0)
    for i in range(RPT // ZR):
        pltpu.async_copy(zbuf, acc_sh.at[pl.ds(row0 + i * ZR, ZR)], gsem0)

    pltpu.sync_copy(ew_hbm.at[wid], ew_all)
    pltpu.sync_copy(dst_hbm.at[wid], dstall)
    for i in range(RPT // ZR):
        pltpu.make_async_copy(zbuf, acc_sh.at[pl.ds(row0 + i * ZR, ZR)],
                              gsem0).wait()
    plsc.subcore_barrier()

    def start_src(c, srcv, sem):
        pltpu.async_copy(src_hbm.at[pl.ds(base + c * C, C)], srcv, sem)

    def wait_src(c, srcv, sem):
        pltpu.make_async_copy(src_hbm.at[pl.ds(base + c * C, C)], srcv,
                              sem).wait()

    def load_dst(c, dstv):
        for gi in range(C // L):
            dstv[pl.ds(gi * L, L)] = dstall[pl.ds(c * C + gi * L, L)]

    def start_gather(rows, srcv, sem):
        pltpu.async_copy(g_hbm.at[srcv], rows, sem)

    def wait_gather(rows, srcv, sem):
        pltpu.make_async_copy(g_hbm.at[srcv], rows, sem).wait()

    lane_sel = [jnp.full((L, 1), j, jnp.int32) for j in range(L)]
    dnums = lax.GatherDimensionNumbers(
        offset_dims=(), collapsed_slice_dims=(0,), start_index_map=(0,))

    def bcast(vec, j):
        return lax.gather(vec, lane_sel[j], dnums, (1,),
                          mode=lax.GatherScatterMode.PROMISE_IN_BOUNDS)

    def scale(c, rows):
        def body(gi, _):
            j0 = gi * L
            ew16 = ew_all[pl.ds(c * C + j0, L)]
            for j in range(L):
                w = ew16[j]
                for kk in range(D // L):
                    rows[j0 + j, pl.ds(kk * L, L)] = (
                        rows[j0 + j, pl.ds(kk * L, L)] * w)
            return 0
        lax.fori_loop(0, C // L, body, 0)

    def start_scatter(rows, dstv, sem):
        pltpu.async_copy(rows, acc_sh.at[dstv], sem, add=True)

    def wait_scatter(rows, dstv, sem):
        pltpu.make_async_copy(rows, acc_sh.at[dstv], sem).wait()

    start_src(0, srcv0, isem0)
    start_src(1, srcv1, isem1)
    wait_src(0, srcv0, isem0)
    start_gather(rows0, srcv0, gsem0)
    wait_src(1, srcv1, isem1)
    start_gather(rows1, srcv1, gsem1)

    def pair(i, _):
        a = 2 * i
        b = a + 1
        wait_gather(rows0, srcv0, gsem0)

        @pl.when(a + 2 < NCHUNK)
        def _():
            start_src(a + 2, srcv0, isem0)
        scale(a, rows0)
        load_dst(a, dstv0)
        start_scatter(rows0, dstv0, ssem0)

        wait_gather(rows1, srcv1, gsem1)

        @pl.when(b + 2 < NCHUNK)
        def _():
            start_src(b + 2, srcv1, isem1)
        scale(b, rows1)
        load_dst(b, dstv1)
        start_scatter(rows1, dstv1, ssem1)

        wait_scatter(rows0, dstv0, ssem0)

        @pl.when(a + 2 < NCHUNK)
        def _():
            wait_src(a + 2, srcv0, isem0)
            start_gather(rows0, srcv0, gsem0)

        wait_scatter(rows1, dstv1, ssem1)

        @pl.when(b + 2 < NCHUNK)
        def _():
            wait_src(b + 2, srcv1, isem1)
            start_gather(rows1, srcv1, gsem1)
        return 0

    lax.fori_loop(0, NCHUNK // 2, pair, 0)

    if NCHUNK % 2 == 1:
        last = NCHUNK - 1
        wait_gather(rows0, srcv0, gsem0)
        scale(last, rows0)
        load_dst(last, dstv0)
        start_scatter(rows0, dstv0, ssem0)
        wait_scatter(rows0, dstv0, ssem0)

    plsc.subcore_barrier()
    pltpu.sync_copy(acc_sh.at[pl.ds(row0, RPT)],
                    out_hbm.at[pl.ds(cid * N2 + row0, RPT)])


def _sc_agg(g, src, dst, ew):
    k = pl.kernel(
        _sc_agg_body,
        out_type=jax.ShapeDtypeStruct((NC * N2, D), jnp.float32),
        mesh=_sc_mesh(),
        scratch_types=[
            pltpu.VMEM_SHARED((N2, D), jnp.float32),
            pltpu.VMEM((EPW,), jnp.int32),
            pltpu.VMEM((EPW,), jnp.float32),
            pltpu.VMEM((C,), jnp.int32),
            pltpu.VMEM((C,), jnp.int32),
            pltpu.VMEM((C,), jnp.int32),
            pltpu.VMEM((C,), jnp.int32),
            pltpu.VMEM((C, D), jnp.float32),
            pltpu.VMEM((C, D), jnp.float32),
            pltpu.VMEM((ZR, D), jnp.float32),
            pltpu.SemaphoreType.DMA,
            pltpu.SemaphoreType.DMA,
            pltpu.SemaphoreType.DMA,
            pltpu.SemaphoreType.DMA,
            pltpu.SemaphoreType.DMA,
            pltpu.SemaphoreType.DMA,
        ],
    )
    return k(g, src, dst.reshape(NW, EPW), ew.reshape(NW, EPW))


BM = 2048


def _tc1_body(d0_ref, d1_ref, x_ref, w_ref, g_ref, dinv_ref):
    s = jnp.sum(d0_ref[...] + d1_ref[...], axis=1, keepdims=True) + 1.0
    dv = lax.rsqrt(s)
    t = jnp.dot(x_ref[...], w_ref[...], preferred_element_type=jnp.float32)
    g_ref[...] = dv * t
    dinv_ref[...] = dv


def _tc1(degp0, degp1, x, W1):
    grid = (pl.cdiv(N, BM),)
    return pl.pallas_call(
        _tc1_body,
        grid=grid,
        in_specs=[
            pl.BlockSpec((BM, L), lambda i: (i, 0)),
            pl.BlockSpec((BM, L), lambda i: (i, 0)),
            pl.BlockSpec((BM, D), lambda i: (i, 0)),
            pl.BlockSpec((D, D), lambda i: (0, 0)),
        ],
        out_specs=[
            pl.BlockSpec((BM, D), lambda i: (i, 0)),
            pl.BlockSpec((BM, 1), lambda i: (i, 0)),
        ],
        out_shape=[
            jax.ShapeDtypeStruct((N, D), jnp.float32),
            jax.ShapeDtypeStruct((N, 1), jnp.float32),
        ],
    )(degp0, degp1, x, W1)


def _tc2_body(p0_ref, p1_ref, g_ref, dv_ref, b_ref, w_ref, g2_ref):
    dv = dv_ref[...]
    h = jnp.maximum(dv * (p0_ref[...] + p1_ref[...] + g_ref[...]) + b_ref[...],
                    0.0)
    t2 = jnp.dot(h, w_ref[...], preferred_element_type=jnp.float32)
    g2_ref[...] = dv * t2


def _tc2(p0, p1, g1, dinv, b1, W2):
    grid = (pl.cdiv(N, BM),)
    return pl.pallas_call(
        _tc2_body,
        grid=grid,
        in_specs=[
            pl.BlockSpec((BM, D), lambda i: (i, 0)),
            pl.BlockSpec((BM, D), lambda i: (i, 0)),
            pl.BlockSpec((BM, D), lambda i: (i, 0)),
            pl.BlockSpec((BM, 1), lambda i: (i, 0)),
            pl.BlockSpec((1, D), lambda i: (0, 0)),
            pl.BlockSpec((D, D), lambda i: (0, 0)),
        ],
        out_specs=pl.BlockSpec((BM, D), lambda i: (i, 0)),
        out_shape=jax.ShapeDtypeStruct((N, D), jnp.float32),
    )(p0, p1, g1, dinv, b1, W2)


def _tc3_body(p0_ref, p1_ref, g_ref, dv_ref, b_ref, out_ref):
    dv = dv_ref[...]
    out_ref[...] = jnp.maximum(
        dv * (p0_ref[...] + p1_ref[...] + g_ref[...]) + b_ref[...], 0.0)


def _tc3(p0, p1, g2, dinv, b2):
    grid = (pl.cdiv(N, BM),)
    return pl.pallas_call(
        _tc3_body,
        grid=grid,
        in_specs=[
            pl.BlockSpec((BM, D), lambda i: (i, 0)),
            pl.BlockSpec((BM, D), lambda i: (i, 0)),
            pl.BlockSpec((BM, D), lambda i: (i, 0)),
            pl.BlockSpec((BM, 1), lambda i: (i, 0)),
            pl.BlockSpec((1, D), lambda i: (0, 0)),
        ],
        out_specs=pl.BlockSpec((BM, D), lambda i: (i, 0)),
        out_shape=jax.ShapeDtypeStruct((N, D), jnp.float32),
    )(p0, p1, g2, dinv, b2)


@jax.jit
def _run(x, edge_index, edge_weight, W1, b1, W2, b2):
    src = edge_index[0]
    dst = edge_index[1]
    b1r = b1.reshape(1, D)
    b2r = b2.reshape(1, D)

    degp = _sc_deg(dst, edge_weight)
    g1, dinv = _tc1(degp[:N], degp[N2:N2 + N], x, W1)

    s1 = _sc_agg(g1, src, dst, edge_weight)
    g2 = _tc2(s1[:N], s1[N2:N2 + N], g1, dinv, b1r, W2)

    s2 = _sc_agg(g2, src, dst, edge_weight)
    h2 = _tc3(s2[:N], s2[N2:N2 + N], g2, dinv, b2r)
    return h2


def kernel(x, edge_index, edge_weight, W1, b1, W2, b2):
    return (_run(x, edge_index, edge_weight, W1, b1, W2, b2), None)


# speedup vs baseline: 22.5428x; 1.0051x over previous
import functools

import jax
import jax.numpy as jnp
from jax import lax
from jax.experimental import pallas as pl
from jax.experimental.pallas import tpu as pltpu
from jax.experimental.pallas import tpu_sc as plsc

N = 10000
E = 320000
D = 128

NC = 2
NS = 16
L = 16
NW = NC * NS
EPW = E // NW
C = 80
NCHUNK = EPW // C
N2 = 10240
RPT = N2 // NS
ZR = 32


def _sc_mesh():
    return plsc.VectorSubcoreMesh(core_axis_name="c", subcore_axis_name="s",
                                  num_cores=NC, num_subcores=NS)


def _sc_deg_body(dst_hbm, ew_hbm, out_hbm, acc_sh, ew_all,
                 dstv0, dstv1, msg0, msg1, zbuf,
                 ssem0, ssem1, isem0, isem1):
    cid = lax.axis_index("c")
    sid = lax.axis_index("s")
    wid = sid * NC + cid
    row0 = sid * RPT

    lanes0 = jnp.zeros((L,), jnp.float32)

    def zb(j, _):
        zbuf[j, :] = lanes0
        return 0
    lax.fori_loop(0, RPT, zb, 0)
    pltpu.sync_copy(zbuf, acc_sh.at[pl.ds(row0, RPT)])
    pltpu.sync_copy(ew_hbm.at[wid], ew_all)
    plsc.subcore_barrier()

    base = wid * EPW
    iota = lax.iota(jnp.int32, L)
    e0 = jnp.where(iota == 0, 1.0, 0.0).astype(jnp.float32)

    def start_idx(c, dstv, sem):
        pltpu.async_copy(dst_hbm.at[pl.ds(base + c * C, C)], dstv, sem)

    def wait_idx(c, dstv, sem):
        pltpu.make_async_copy(dst_hbm.at[pl.ds(base + c * C, C)], dstv,
                              sem).wait()

    def build(c, msg):
        def mrow(gi, _):
            j0 = gi * L
            ew16 = ew_all[pl.ds(c * C + j0, L)]
            for j in range(L):
                msg[j0 + j, :] = ew16[j] * e0
            return 0
        lax.fori_loop(0, C // L, mrow, 0)

    def start_scatter(msg, dstv, sem):
        pltpu.async_copy(msg, acc_sh.at[dstv], sem, add=True)

    def wait_scatter(msg, dstv, sem):
        pltpu.make_async_copy(msg, acc_sh.at[dstv], sem).wait()

    start_idx(0, dstv0, isem0)
    start_idx(1, dstv1, isem1)

    def pair(i, _):
        a = 2 * i
        b = a + 1
        build(a, msg0)
        wait_idx(a, dstv0, isem0)
        start_scatter(msg0, dstv0, ssem0)
        build(b, msg1)
        wait_idx(b, dstv1, isem1)
        start_scatter(msg1, dstv1, ssem1)
        wait_scatter(msg0, dstv0, ssem0)

        @pl.when(a + 2 < NCHUNK)
        def _():
            start_idx(a + 2, dstv0, isem0)
        wait_scatter(msg1, dstv1, ssem1)

        @pl.when(b + 2 < NCHUNK)
        def _():
            start_idx(b + 2, dstv1, isem1)
        return 0

    lax.fori_loop(0, NCHUNK // 2, pair, 0)
    if NCHUNK % 2 == 1:
        last = NCHUNK - 1
        build(last, msg0)
        wait_idx(last, dstv0, isem0)
        start_scatter(msg0, dstv0, ssem0)
        wait_scatter(msg0, dstv0, ssem0)

    plsc.subcore_barrier()
    pltpu.sync_copy(acc_sh.at[pl.ds(row0, RPT)],
                    out_hbm.at[pl.ds(cid * N2 + row0, RPT)])


def _sc_deg(dst, ew):
    k = pl.kernel(
        _sc_deg_body,
        out_type=jax.ShapeDtypeStruct((NC * N2, L), jnp.float32),
        mesh=_sc_mesh(),
        scratch_types=[
            pltpu.VMEM_SHARED((N2, L), jnp.float32),
            pltpu.VMEM((EPW,), jnp.float32),
            pltpu.VMEM((C,), jnp.int32),
            pltpu.VMEM((C,), jnp.int32),
            pltpu.VMEM((C, L), jnp.float32),
            pltpu.VMEM((C, L), jnp.float32),
            pltpu.VMEM((RPT, L), jnp.float32),
            pltpu.SemaphoreType.DMA,
            pltpu.SemaphoreType.DMA,
            pltpu.SemaphoreType.DMA,
            pltpu.SemaphoreType.DMA,
        ],
    )
    return k(dst, ew.reshape(NW, EPW))


def _sc_agg_body(g_hbm, src_hbm, dst_hbm, ew_hbm, out_hbm,
                 acc_sh, dstall, ew_all,
                 srcv0, srcv1, dstv0, dstv1,
                 rows0, rows1, zbuf,
                 gsem0, gsem1, ssem0, ssem1, isem0, isem1):
    cid = lax.axis_index("c")
    sid = lax.axis_index("s")
    wid = sid * NC + cid
    base = wid * EPW
    row0 = sid * RPT

    lanes0 = jnp.zeros((L,), jnp.float32)

    def zb(j, _):
        for kk in range(D // L):
            zbuf[j, pl.ds(kk * L, L)] = lanes0
        return 0
    lax.fori_loop(0, ZR, zb, 0)
    for i in range(RPT // ZR):
        pltpu.async_copy(zbuf, acc_sh.at[pl.ds(row0 + i * ZR, ZR)], gsem0)

    pltpu.sync_copy(ew_hbm.at[wid], ew_all)
    pltpu.sync_copy(dst_hbm.at[wid], dstall)
    for i in range(RPT // ZR):
        pltpu.make_async_copy(zbuf, acc_sh.at[pl.ds(row0 + i * ZR, ZR)],
                              gsem0).wait()
    plsc.subcore_barrier()

    def start_src(c, srcv, sem):
        pltpu.async_copy(src_hbm.at[pl.ds(base + c * C, C)], srcv, sem)

    def wait_src(c, srcv, sem):
        pltpu.make_async_copy(src_hbm.at[pl.ds(base + c * C, C)], srcv,
                              sem).wait()

    def load_dst(c, dstv):
        for gi in range(C // L):
            dstv[pl.ds(gi * L, L)] = dstall[pl.ds(c * C + gi * L, L)]

    def start_gather(rows, srcv, sem):
        pltpu.async_copy(g_hbm.at[srcv], rows, sem)

    def wait_gather(rows, srcv, sem):
        pltpu.make_async_copy(g_hbm.at[srcv], rows, sem).wait()

    lane_sel = [jnp.full((L, 1), j, jnp.int32) for j in range(L)]
    dnums = lax.GatherDimensionNumbers(
        offset_dims=(), collapsed_slice_dims=(0,), start_index_map=(0,))

    def bcast(vec, j):
        return lax.gather(vec, lane_sel[j], dnums, (1,),
                          mode=lax.GatherScatterMode.PROMISE_IN_BOUNDS)

    def scale(c, rows):
        def body(gi, _):
            j0 = gi * L
            ew16 = ew_all[pl.ds(c * C + j0, L)]
            for j in range(L):
                w = ew16[j]
                for kk in range(D // L):
                    rows[j0 + j, pl.ds(kk * L, L)] = (
                        rows[j0 + j, pl.ds(kk * L, L)] * w)
            return 0
        lax.fori_loop(0, C // L, body, 0)

    def start_scatter(rows, dstv, sem):
        pltpu.async_copy(rows, acc_sh.at[dstv], sem, add=True)

    def wait_scatter(rows, dstv, sem):
        pltpu.make_async_copy(rows, acc_sh.at[dstv], sem).wait()

    start_src(0, srcv0, isem0)
    start_src(1, srcv1, isem1)
    wait_src(0, srcv0, isem0)
    start_gather(rows0, srcv0, gsem0)
    wait_src(1, srcv1, isem1)
    start_gather(rows1, srcv1, gsem1)

    def pair(i, _):
        a = 2 * i
        b = a + 1
        wait_gather(rows0, srcv0, gsem0)

        @pl.when(a + 2 < NCHUNK)
        def _():
            start_src(a + 2, srcv0, isem0)
        scale(a, rows0)
        load_dst(a, dstv0)
        start_scatter(rows0, dstv0, ssem0)

        wait_gather(rows1, srcv1, gsem1)

        @pl.when(b + 2 < NCHUNK)
        def _():
            start_src(b + 2, srcv1, isem1)
        scale(b, rows1)
        load_dst(b, dstv1)
        start_scatter(rows1, dstv1, ssem1)

        wait_scatter(rows0, dstv0, ssem0)

        @pl.when(a + 2 < NCHUNK)
        def _():
            wait_src(a + 2, srcv0, isem0)
            start_gather(rows0, srcv0, gsem0)

        wait_scatter(rows1, dstv1, ssem1)

        @pl.when(b + 2 < NCHUNK)
        def _():
            wait_src(b + 2, srcv1, isem1)
            start_gather(rows1, srcv1, gsem1)
        return 0

    lax.fori_loop(0, NCHUNK // 2, pair, 0)

    if NCHUNK % 2 == 1:
        last = NCHUNK - 1
        wait_gather(rows0, srcv0, gsem0)
        scale(last, rows0)
        load_dst(last, dstv0)
        start_scatter(rows0, dstv0, ssem0)
        wait_scatter(rows0, dstv0, ssem0)

    plsc.subcore_barrier()
    pltpu.sync_copy(acc_sh.at[pl.ds(row0, RPT)],
                    out_hbm.at[pl.ds(cid * N2 + row0, RPT)])


def _sc_agg(g, src, dst, ew):
    k = pl.kernel(
        _sc_agg_body,
        out_type=jax.ShapeDtypeStruct((NC * N2, D), jnp.float32),
        mesh=_sc_mesh(),
        scratch_types=[
            pltpu.VMEM_SHARED((N2, D), jnp.float32),
            pltpu.VMEM((EPW,), jnp.int32),
            pltpu.VMEM((EPW,), jnp.float32),
            pltpu.VMEM((C,), jnp.int32),
            pltpu.VMEM((C,), jnp.int32),
            pltpu.VMEM((C,), jnp.int32),
            pltpu.VMEM((C,), jnp.int32),
            pltpu.VMEM((C, D), jnp.float32),
            pltpu.VMEM((C, D), jnp.float32),
            pltpu.VMEM((ZR, D), jnp.float32),
            pltpu.SemaphoreType.DMA,
            pltpu.SemaphoreType.DMA,
            pltpu.SemaphoreType.DMA,
            pltpu.SemaphoreType.DMA,
            pltpu.SemaphoreType.DMA,
            pltpu.SemaphoreType.DMA,
        ],
    )
    return k(g, src, dst.reshape(NW, EPW), ew.reshape(NW, EPW))


BM = 5120


def _tc1_body(d0_ref, d1_ref, x_ref, w_ref, g_ref, dinv_ref):
    s = jnp.sum(d0_ref[...] + d1_ref[...], axis=1, keepdims=True) + 1.0
    dv = lax.rsqrt(s)
    t = jnp.dot(x_ref[...], w_ref[...], preferred_element_type=jnp.float32)
    g_ref[...] = dv * t
    dinv_ref[...] = dv


def _tc1(degp0, degp1, x, W1):
    grid = (pl.cdiv(N, BM),)
    return pl.pallas_call(
        _tc1_body,
        grid=grid,
        in_specs=[
            pl.BlockSpec((BM, L), lambda i: (i, 0)),
            pl.BlockSpec((BM, L), lambda i: (i, 0)),
            pl.BlockSpec((BM, D), lambda i: (i, 0)),
            pl.BlockSpec((D, D), lambda i: (0, 0)),
        ],
        out_specs=[
            pl.BlockSpec((BM, D), lambda i: (i, 0)),
            pl.BlockSpec((BM, 1), lambda i: (i, 0)),
        ],
        out_shape=[
            jax.ShapeDtypeStruct((N, D), jnp.float32),
            jax.ShapeDtypeStruct((N, 1), jnp.float32),
        ],
    )(degp0, degp1, x, W1)


def _tc2_body(p0_ref, p1_ref, g_ref, dv_ref, b_ref, w_ref, g2_ref):
    dv = dv_ref[...]
    h = jnp.maximum(dv * (p0_ref[...] + p1_ref[...] + g_ref[...]) + b_ref[...],
                    0.0)
    t2 = jnp.dot(h, w_ref[...], preferred_element_type=jnp.float32)
    g2_ref[...] = dv * t2


def _tc2(p0, p1, g1, dinv, b1, W2):
    grid = (pl.cdiv(N, BM),)
    return pl.pallas_call(
        _tc2_body,
        grid=grid,
        in_specs=[
            pl.BlockSpec((BM, D), lambda i: (i, 0)),
            pl.BlockSpec((BM, D), lambda i: (i, 0)),
            pl.BlockSpec((BM, D), lambda i: (i, 0)),
            pl.BlockSpec((BM, 1), lambda i: (i, 0)),
            pl.BlockSpec((1, D), lambda i: (0, 0)),
            pl.BlockSpec((D, D), lambda i: (0, 0)),
        ],
        out_specs=pl.BlockSpec((BM, D), lambda i: (i, 0)),
        out_shape=jax.ShapeDtypeStruct((N, D), jnp.float32),
    )(p0, p1, g1, dinv, b1, W2)


def _tc3_body(p0_ref, p1_ref, g_ref, dv_ref, b_ref, out_ref):
    dv = dv_ref[...]
    out_ref[...] = jnp.maximum(
        dv * (p0_ref[...] + p1_ref[...] + g_ref[...]) + b_ref[...], 0.0)


def _tc3(p0, p1, g2, dinv, b2):
    grid = (pl.cdiv(N, BM),)
    return pl.pallas_call(
        _tc3_body,
        grid=grid,
        in_specs=[
            pl.BlockSpec((BM, D), lambda i: (i, 0)),
            pl.BlockSpec((BM, D), lambda i: (i, 0)),
            pl.BlockSpec((BM, D), lambda i: (i, 0)),
            pl.BlockSpec((BM, 1), lambda i: (i, 0)),
            pl.BlockSpec((1, D), lambda i: (0, 0)),
        ],
        out_specs=pl.BlockSpec((BM, D), lambda i: (i, 0)),
        out_shape=jax.ShapeDtypeStruct((N, D), jnp.float32),
    )(p0, p1, g2, dinv, b2)


@jax.jit
def _run(x, edge_index, edge_weight, W1, b1, W2, b2):
    src = edge_index[0]
    dst = edge_index[1]
    b1r = b1.reshape(1, D)
    b2r = b2.reshape(1, D)

    degp = _sc_deg(dst, edge_weight)
    g1, dinv = _tc1(degp[:N], degp[N2:N2 + N], x, W1)

    s1 = _sc_agg(g1, src, dst, edge_weight)
    g2 = _tc2(s1[:N], s1[N2:N2 + N], g1, dinv, b1r, W2)

    s2 = _sc_agg(g2, src, dst, edge_weight)
    h2 = _tc3(s2[:N], s2[N2:N2 + N], g2, dinv, b2r)
    return h2


def kernel(x, edge_index, edge_weight, W1, b1, W2, b2):
    return (_run(x, edge_index, edge_weight, W1, b1, W2, b2), None)
